# baseline (speedup 1.0000x reference)
import jax
import jax.numpy as jnp
from jax.experimental import pallas as pl
from jax.experimental.pallas import tpu as pltpu

_BR = 8000
_BC = 128


def _fused_mlp_kernel(x_ref, w1_ref, b1_ref, w2_ref, b2_ref, o_ref):
    h = jax.lax.dot_general(
        x_ref[...], w1_ref[...],
        dimension_numbers=(((1,), (0,)), ((), ())),
        preferred_element_type=jnp.float32,
    )
    h = jnp.maximum(h + b1_ref[...], 0.0)
    o = jax.lax.dot_general(
        h, w2_ref[...],
        dimension_numbers=(((1,), (0,)), ((), ())),
        preferred_element_type=jnp.float32,
    )
    o_ref[...] = o + b2_ref[...]


def kernel(x_transaction, x_merchant, x_category, edge_index_belongs_to, edge_index_has_category, W_enc_t, b_enc_t, W_enc_m, b_enc_m, W_enc_c, b_enc_c, lin_l_bm_0, bias_bm_0, lin_r_bm_0, lin_l_tc_0, bias_tc_0, lin_r_tc_0, lin_l_bm_1, bias_bm_1, lin_r_bm_1, lin_l_tc_1, bias_tc_1, lin_r_tc_1, W_cls, b_cls):
    NT, D = x_transaction.shape
    H = W_enc_t.shape[1]
    OUT = W_cls.shape[1]

    grid = (NT // _BR, pl.cdiv(OUT, _BC))

    b1 = b_enc_t.reshape(1, H)
    b2 = b_cls.reshape(1, OUT)

    return pl.pallas_call(
        _fused_mlp_kernel,
        grid=grid,
        in_specs=[
            pl.BlockSpec((_BR, D), lambda i, j: (i, 0)),
            pl.BlockSpec((D, H), lambda i, j: (0, 0)),
            pl.BlockSpec((1, H), lambda i, j: (0, 0)),
            pl.BlockSpec((H, _BC), lambda i, j: (0, j)),
            pl.BlockSpec((1, _BC), lambda i, j: (0, j)),
        ],
        out_specs=pl.BlockSpec((_BR, _BC), lambda i, j: (i, j)),
        out_shape=jax.ShapeDtypeStruct((NT, OUT), jnp.float32),
        compiler_params=pltpu.CompilerParams(
            dimension_semantics=("parallel", "arbitrary"),
        ),
    )(x_transaction, W_enc_t, b1, W_cls, b2)


# speedup vs baseline: 1.1389x; 1.1389x over previous
import jax
import jax.numpy as jnp
from jax.experimental import pallas as pl
from jax.experimental.pallas import tpu as pltpu


def _fused_mlp_kernel(x_ref, w1_ref, b1_ref, w2_ref, b2_ref, o_ref):
    h = jax.lax.dot_general(
        x_ref[...], w1_ref[...],
        dimension_numbers=(((1,), (0,)), ((), ())),
        preferred_element_type=jnp.float32,
    )
    h = jnp.maximum(h + b1_ref[...], 0.0)
    o = jax.lax.dot_general(
        h, w2_ref[...],
        dimension_numbers=(((1,), (0,)), ((), ())),
        preferred_element_type=jnp.float32,
    )
    o_ref[...] = o + b2_ref[...]


def kernel(x_transaction, x_merchant, x_category, edge_index_belongs_to, edge_index_has_category, W_enc_t, b_enc_t, W_enc_m, b_enc_m, W_enc_c, b_enc_c, lin_l_bm_0, bias_bm_0, lin_r_bm_0, lin_l_tc_0, bias_tc_0, lin_r_tc_0, lin_l_bm_1, bias_bm_1, lin_r_bm_1, lin_l_tc_1, bias_tc_1, lin_r_tc_1, W_cls, b_cls):
    NT, D = x_transaction.shape
    H = W_enc_t.shape[1]
    OUT = W_cls.shape[1]

    BR = 8000
    grid = (pl.cdiv(NT, BR),)

    b1 = b_enc_t.reshape(1, H)
    b2 = b_cls.reshape(1, OUT)

    return pl.pallas_call(
        _fused_mlp_kernel,
        grid=grid,
        in_specs=[
            pl.BlockSpec((BR, D), lambda i: (i, 0)),
            pl.BlockSpec((D, H), lambda i: (0, 0)),
            pl.BlockSpec((1, H), lambda i: (0, 0)),
            pl.BlockSpec((H, OUT), lambda i: (0, 0)),
            pl.BlockSpec((1, OUT), lambda i: (0, 0)),
        ],
        out_specs=pl.BlockSpec((BR, OUT), lambda i: (i, 0)),
        out_shape=jax.ShapeDtypeStruct((NT, OUT), jnp.float32),
        compiler_params=pltpu.CompilerParams(
            dimension_semantics=("parallel",),
        ),
    )(x_transaction, W_enc_t, b1, W_cls, b2)
